# baseline (speedup 1.0000x reference)
import functools

import jax
import jax.numpy as jnp
from jax import lax
from jax.experimental import pallas as pl
from jax.experimental.pallas import tpu as pltpu
from jax.experimental.pallas import tpu_sc as plsc

N = 100000
E = 3200000
D = 16
R = 3
OUT_A = 2
OUT_B = 16

NC = 2
NS = 16
NT = NC * NS
CHUNK = 768
RPC = CHUNK // 64
NCHUNK = 132
EPT = NCHUNK * CHUNK
EPAD = NT * EPT
ESLACK = 0
NPAD = 100096
NPT = NPAD // NS

BN = 10000


def _sc_edge_body(table, gidx, dstx, normx, out, acc,
                  i0, i1, d0, d1, n0, n1,
                  rows0, rows1, lsem, gsem, ssem):
    c = lax.axis_index("c")
    s = lax.axis_index("s")
    w = c * NS + s
    idxr = (i0, i1)
    dstr = (d0, d1)
    normr = (n0, n1)
    rowsr = (rows0, rows1)

    def fire_linear(k, pr):
        r0 = w * (EPT // 64) + k * RPC
        e0 = w * EPT + k * CHUNK
        pltpu.async_copy(gidx.at[pl.ds(r0, RPC)], idxr[pr], lsem)
        pltpu.async_copy(dstx.at[pl.ds(r0, RPC)], dstr[pr], lsem)
        pltpu.async_copy(normx.at[pl.ds(e0, CHUNK)], normr[pr], lsem)

    def wait_linear(k, pr):
        r0 = w * (EPT // 64) + k * RPC
        e0 = w * EPT + k * CHUNK
        pltpu.make_async_copy(gidx.at[pl.ds(r0, RPC)], idxr[pr], lsem).wait()
        pltpu.make_async_copy(dstx.at[pl.ds(r0, RPC)], dstr[pr], lsem).wait()
        pltpu.make_async_copy(normx.at[pl.ds(e0, CHUNK)], normr[pr],
                              lsem).wait()

    def fire_gather(pr):
        for j in range(RPC):
            pltpu.async_copy(table.at[idxr[pr].at[j]],
                             rowsr[pr].at[pl.ds(j * 64, 64)], gsem)

    def wait_gather(pr):
        for j in range(RPC):
            pltpu.make_async_copy(table.at[idxr[pr].at[j]],
                                  rowsr[pr].at[pl.ds(j * 64, 64)],
                                  gsem).wait()

    def scale(pr):
        @pl.loop(0, CHUNK // 16)
        def _scale(g):
            nv = normr[pr][pl.ds(g * 16, 16)]
            for j in range(16):
                e = g * 16 + j
                rowsr[pr][e, :] = rowsr[pr][e, :] * nv[j]

    def fire_scatter(pr):
        for j in range(RPC):
            pltpu.async_copy(rowsr[pr].at[pl.ds(j * 64, 64)],
                             acc.at[dstr[pr].at[j]], ssem, add=True)

    def wait_scatter(pr):
        for j in range(RPC):
            pltpu.make_async_copy(rowsr[pr].at[pl.ds(j * 64, 64)],
                                  acc.at[dstr[pr].at[j]], ssem).wait()

    @pl.loop(0, CHUNK)
    def _zero(i):
        rows0[i, :] = jnp.zeros((D,), jnp.float32)

    @pl.loop(0, 8)
    def _zacc(k):
        pltpu.sync_copy(rows0, acc.at[pl.ds(s * NPT + k * CHUNK, CHUNK)])
    pltpu.sync_copy(rows0.at[pl.ds(0, NPT - 8 * CHUNK)],
                    acc.at[pl.ds(s * NPT + 8 * CHUNK, NPT - 8 * CHUNK)])

    plsc.subcore_barrier()

    def sub(i, pr):
        wait_linear(i, pr)
        fire_gather(pr)
        wait_scatter(1 - pr)
        fire_linear(i + 1, 1 - pr)
        wait_gather(pr)
        scale(pr)
        fire_scatter(pr)

    fire_linear(0, 0)
    wait_linear(0, 0)
    fire_gather(0)
    fire_linear(1, 1)
    wait_gather(0)
    scale(0)
    fire_scatter(0)

    @pl.loop(0, (NCHUNK - 2) // 2)
    def _mid(t):
        sub(1 + 2 * t, 1)
        sub(2 + 2 * t, 0)

    wait_linear(NCHUNK - 1, 1)
    fire_gather(1)
    wait_scatter(0)
    wait_gather(1)
    scale(1)
    fire_scatter(1)
    wait_scatter(1)

    plsc.subcore_barrier()
    pltpu.sync_copy(acc.at[pl.ds(s * NPT, NPT)],
                    out.at[pl.ds(c * NPAD + s * NPT, NPT)])


_sc_edge = pl.kernel(
    _sc_edge_body,
    out_type=jax.ShapeDtypeStruct((2 * NPAD, D), jnp.float32),
    mesh=plsc.VectorSubcoreMesh(core_axis_name="c", subcore_axis_name="s",
                                num_cores=NC, num_subcores=NS),
    scratch_types=[
        pltpu.MemorySpace.VMEM_SHARED((NPAD, D), jnp.float32),
        *[pltpu.VMEM((RPC, 64), jnp.int32) for _ in range(2)],
        *[pltpu.VMEM((RPC, 64), jnp.int32) for _ in range(2)],
        *[pltpu.VMEM((CHUNK,), jnp.float32) for _ in range(2)],
        *[pltpu.VMEM((CHUNK, D), jnp.float32) for _ in range(2)],
        *[pltpu.SemaphoreType.DMA for _ in range(3)],
    ],
    compiler_params=pltpu.CompilerParams(use_tc_tiling_on_sc=False),
)


def _transform_body(x_ref, w_ref, xw_ref):
    xw_ref[...] = jnp.dot(x_ref[...], w_ref[...],
                          preferred_element_type=jnp.float32)


def _tc_transform(x, wc):
    return pl.pallas_call(
        _transform_body,
        grid=(N // BN,),
        in_specs=[
            pl.BlockSpec((BN, D), lambda i: (i, 0)),
            pl.BlockSpec((D, R * D), lambda i: (0, 0)),
        ],
        out_specs=pl.BlockSpec((BN, R * D), lambda i: (i, 0)),
        out_shape=jax.ShapeDtypeStruct((N, R * D), jnp.float32),
    )(x, wc)


def _combine_body(with_skip, *refs):
    if with_skip:
        p0_ref, p1_ref, h_ref, w_ref, hn_ref, xw_ref = refs
        h = jnp.maximum(p0_ref[...] + p1_ref[...] + h_ref[...], 0.0)
    else:
        p0_ref, p1_ref, w_ref, hn_ref, xw_ref = refs
        h = jnp.maximum(p0_ref[...] + p1_ref[...], 0.0)
    hn_ref[...] = h
    xw_ref[...] = jnp.dot(h, w_ref[...], preferred_element_type=jnp.float32)


def _tc_combine(p0, p1, hprev, wc):
    with_skip = hprev is not None
    hb = [pl.BlockSpec((BN, D), lambda i: (i, 0))] if with_skip else []
    ops = (p0, p1) + ((hprev,) if with_skip else ()) + (wc,)
    return pl.pallas_call(
        functools.partial(_combine_body, with_skip),
        grid=(N // BN,),
        in_specs=[
            pl.BlockSpec((BN, D), lambda i: (i, 0)),
            pl.BlockSpec((BN, D), lambda i: (i, 0)),
            *hb,
            pl.BlockSpec((D, R * D), lambda i: (0, 0)),
        ],
        out_specs=[
            pl.BlockSpec((BN, D), lambda i: (i, 0)),
            pl.BlockSpec((BN, R * D), lambda i: (i, 0)),
        ],
        out_shape=[
            jax.ShapeDtypeStruct((N, D), jnp.float32),
            jax.ShapeDtypeStruct((N, R * D), jnp.float32),
        ],
    )(*ops)


def _head_body(p0_ref, p1_ref, h_ref, w_ref, b_ref, o_ref):
    h = jnp.maximum(p0_ref[...] + p1_ref[...] + h_ref[...], 0.0)
    o_ref[...] = jnp.dot(h, w_ref[...],
                         preferred_element_type=jnp.float32) + b_ref[...]


def _tc_head(p0, p1, hprev, wh, bh):
    no = OUT_A + OUT_B
    return pl.pallas_call(
        _head_body,
        grid=(N // BN,),
        in_specs=[
            pl.BlockSpec((BN, D), lambda i: (i, 0)),
            pl.BlockSpec((BN, D), lambda i: (i, 0)),
            pl.BlockSpec((BN, D), lambda i: (i, 0)),
            pl.BlockSpec((D, no), lambda i: (0, 0)),
            pl.BlockSpec((1, no), lambda i: (0, 0)),
        ],
        out_specs=pl.BlockSpec((BN, no), lambda i: (i, 0)),
        out_shape=jax.ShapeDtypeStruct((N, no), jnp.float32),
    )(p0, p1, hprev, wh, bh)


def kernel(x, edge_index, rel_type, norm, W1, W2, W3, Wa, ba, Wb, bb):
    src = edge_index[0].astype(jnp.int32)
    dst = edge_index[1].astype(jnp.int32)
    rel = rel_type.astype(jnp.int32)
    gidx = src * R + rel
    pad = EPAD + ESLACK - E
    zi = jnp.zeros((pad,), jnp.int32)
    gidx2 = jnp.concatenate([gidx, zi]).reshape((EPAD + ESLACK) // 64, 64)
    dst2 = jnp.concatenate([dst, zi]).reshape((EPAD + ESLACK) // 64, 64)
    normp = jnp.concatenate([norm, jnp.zeros((pad,), jnp.float32)])

    wc1 = W1.transpose(1, 0, 2).reshape(D, R * D)
    wc2 = W2.transpose(1, 0, 2).reshape(D, R * D)
    wc3 = W3.transpose(1, 0, 2).reshape(D, R * D)
    wh = jnp.concatenate([Wa.T, Wb.T], axis=1)
    bh = jnp.concatenate([ba, bb]).reshape(1, OUT_A + OUT_B)

    xw1 = _tc_transform(x, wc1)
    p1 = _sc_edge(xw1.reshape(R * N, D), gidx2, dst2, normp)
    h1, xw2 = _tc_combine(p1[:N], p1[NPAD:NPAD + N], None, wc2)
    p2 = _sc_edge(xw2.reshape(R * N, D), gidx2, dst2, normp)
    h2, xw3 = _tc_combine(p2[:N], p2[NPAD:NPAD + N], h1, wc3)
    p3 = _sc_edge(xw3.reshape(R * N, D), gidx2, dst2, normp)
    out = _tc_head(p3[:N], p3[NPAD:NPAD + N], h2, wh, bh)
    return out[:, :OUT_A], out[:, OUT_A:]


# speedup vs baseline: 1.0467x; 1.0467x over previous
import functools

import jax
import jax.numpy as jnp
from jax import lax
from jax.experimental import pallas as pl
from jax.experimental.pallas import tpu as pltpu
from jax.experimental.pallas import tpu_sc as plsc

N = 100000
E = 3200000
D = 16
R = 3
OUT_A = 2
OUT_B = 16

NC = 2
NS = 16
NT = NC * NS
CHUNK = 768
RPC = CHUNK // 64
NCHUNK = 132
EPT = NCHUNK * CHUNK
EPAD = NT * EPT
ESLACK = 0
NPAD = 100096
NPT = NPAD // NS

BN = 10000


def _sc_edge_body(table, gidx, dstx, normx, out, acc,
                  i0, i1, d0, d1, n0, n1,
                  rows0, rows1, lsem, g0, g1, g2, g3, ssem):
    c = lax.axis_index("c")
    s = lax.axis_index("s")
    w = c * NS + s
    idxr = (i0, i1)
    dstr = (d0, d1)
    normr = (n0, n1)
    rowsr = (rows0, rows1)
    gsems = (g0, g1, g2, g3)

    def fire_linear(k, pr):
        r0 = w * (EPT // 64) + k * RPC
        e0 = w * EPT + k * CHUNK
        pltpu.async_copy(gidx.at[pl.ds(r0, RPC)], idxr[pr], lsem)
        pltpu.async_copy(dstx.at[pl.ds(r0, RPC)], dstr[pr], lsem)
        pltpu.async_copy(normx.at[pl.ds(e0, CHUNK)], normr[pr], lsem)

    def wait_linear(k, pr):
        r0 = w * (EPT // 64) + k * RPC
        e0 = w * EPT + k * CHUNK
        pltpu.make_async_copy(gidx.at[pl.ds(r0, RPC)], idxr[pr], lsem).wait()
        pltpu.make_async_copy(dstx.at[pl.ds(r0, RPC)], dstr[pr], lsem).wait()
        pltpu.make_async_copy(normx.at[pl.ds(e0, CHUNK)], normr[pr],
                              lsem).wait()

    QS = RPC // 4

    def fire_gather(pr):
        for j in range(RPC):
            pltpu.async_copy(table.at[idxr[pr].at[j]],
                             rowsr[pr].at[pl.ds(j * 64, 64)], gsems[j // QS])

    def wait_gather_q(pr, q):
        for j in range(q * QS, (q + 1) * QS):
            pltpu.make_async_copy(table.at[idxr[pr].at[j]],
                                  rowsr[pr].at[pl.ds(j * 64, 64)],
                                  gsems[j // QS]).wait()

    def scale_q(pr, q):
        sub = QS * 64 // 16

        @pl.loop(0, sub)
        def _scale(g):
            e0 = q * QS * 64 + g * 16
            nv = normr[pr][pl.ds(e0, 16)]
            for j in range(16):
                e = e0 + j
                rowsr[pr][e, :] = rowsr[pr][e, :] * nv[j]

    def wait_scale_gather(pr):
        for q in range(4):
            wait_gather_q(pr, q)
            scale_q(pr, q)

    def fire_scatter(pr):
        for j in range(RPC):
            pltpu.async_copy(rowsr[pr].at[pl.ds(j * 64, 64)],
                             acc.at[dstr[pr].at[j]], ssem, add=True)

    def wait_scatter(pr):
        for j in range(RPC):
            pltpu.make_async_copy(rowsr[pr].at[pl.ds(j * 64, 64)],
                                  acc.at[dstr[pr].at[j]], ssem).wait()

    @pl.loop(0, CHUNK)
    def _zero(i):
        rows0[i, :] = jnp.zeros((D,), jnp.float32)

    @pl.loop(0, 8)
    def _zacc(k):
        pltpu.sync_copy(rows0, acc.at[pl.ds(s * NPT + k * CHUNK, CHUNK)])
    pltpu.sync_copy(rows0.at[pl.ds(0, NPT - 8 * CHUNK)],
                    acc.at[pl.ds(s * NPT + 8 * CHUNK, NPT - 8 * CHUNK)])

    plsc.subcore_barrier()

    def sub(i, pr):
        wait_linear(i, pr)
        fire_gather(pr)
        wait_scatter(1 - pr)
        fire_linear(i + 1, 1 - pr)
        wait_scale_gather(pr)
        fire_scatter(pr)

    fire_linear(0, 0)
    wait_linear(0, 0)
    fire_gather(0)
    fire_linear(1, 1)
    wait_scale_gather(0)
    fire_scatter(0)

    @pl.loop(0, (NCHUNK - 2) // 2)
    def _mid(t):
        sub(1 + 2 * t, 1)
        sub(2 + 2 * t, 0)

    wait_linear(NCHUNK - 1, 1)
    fire_gather(1)
    wait_scatter(0)
    wait_scale_gather(1)
    fire_scatter(1)
    wait_scatter(1)

    plsc.subcore_barrier()
    pltpu.sync_copy(acc.at[pl.ds(s * NPT, NPT)],
                    out.at[pl.ds(c * NPAD + s * NPT, NPT)])


_sc_edge = pl.kernel(
    _sc_edge_body,
    out_type=jax.ShapeDtypeStruct((2 * NPAD, D), jnp.float32),
    mesh=plsc.VectorSubcoreMesh(core_axis_name="c", subcore_axis_name="s",
                                num_cores=NC, num_subcores=NS),
    scratch_types=[
        pltpu.MemorySpace.VMEM_SHARED((NPAD, D), jnp.float32),
        *[pltpu.VMEM((RPC, 64), jnp.int32) for _ in range(2)],
        *[pltpu.VMEM((RPC, 64), jnp.int32) for _ in range(2)],
        *[pltpu.VMEM((CHUNK,), jnp.float32) for _ in range(2)],
        *[pltpu.VMEM((CHUNK, D), jnp.float32) for _ in range(2)],
        *[pltpu.SemaphoreType.DMA for _ in range(6)],
    ],
    compiler_params=pltpu.CompilerParams(use_tc_tiling_on_sc=False),
)


def _transform_body(x_ref, w_ref, xw_ref):
    xw_ref[...] = jnp.dot(x_ref[...], w_ref[...],
                          preferred_element_type=jnp.float32)


def _tc_transform(x, wc):
    return pl.pallas_call(
        _transform_body,
        grid=(N // BN,),
        in_specs=[
            pl.BlockSpec((BN, D), lambda i: (i, 0)),
            pl.BlockSpec((D, R * D), lambda i: (0, 0)),
        ],
        out_specs=pl.BlockSpec((BN, R * D), lambda i: (i, 0)),
        out_shape=jax.ShapeDtypeStruct((N, R * D), jnp.float32),
    )(x, wc)


def _combine_body(with_skip, *refs):
    if with_skip:
        p0_ref, p1_ref, h_ref, w_ref, hn_ref, xw_ref = refs
        h = jnp.maximum(p0_ref[...] + p1_ref[...] + h_ref[...], 0.0)
    else:
        p0_ref, p1_ref, w_ref, hn_ref, xw_ref = refs
        h = jnp.maximum(p0_ref[...] + p1_ref[...], 0.0)
    hn_ref[...] = h
    xw_ref[...] = jnp.dot(h, w_ref[...], preferred_element_type=jnp.float32)


def _tc_combine(p0, p1, hprev, wc):
    with_skip = hprev is not None
    hb = [pl.BlockSpec((BN, D), lambda i: (i, 0))] if with_skip else []
    ops = (p0, p1) + ((hprev,) if with_skip else ()) + (wc,)
    return pl.pallas_call(
        functools.partial(_combine_body, with_skip),
        grid=(N // BN,),
        in_specs=[
            pl.BlockSpec((BN, D), lambda i: (i, 0)),
            pl.BlockSpec((BN, D), lambda i: (i, 0)),
            *hb,
            pl.BlockSpec((D, R * D), lambda i: (0, 0)),
        ],
        out_specs=[
            pl.BlockSpec((BN, D), lambda i: (i, 0)),
            pl.BlockSpec((BN, R * D), lambda i: (i, 0)),
        ],
        out_shape=[
            jax.ShapeDtypeStruct((N, D), jnp.float32),
            jax.ShapeDtypeStruct((N, R * D), jnp.float32),
        ],
    )(*ops)


def _head_body(p0_ref, p1_ref, h_ref, w_ref, b_ref, o_ref):
    h = jnp.maximum(p0_ref[...] + p1_ref[...] + h_ref[...], 0.0)
    o_ref[...] = jnp.dot(h, w_ref[...],
                         preferred_element_type=jnp.float32) + b_ref[...]


def _tc_head(p0, p1, hprev, wh, bh):
    no = OUT_A + OUT_B
    return pl.pallas_call(
        _head_body,
        grid=(N // BN,),
        in_specs=[
            pl.BlockSpec((BN, D), lambda i: (i, 0)),
            pl.BlockSpec((BN, D), lambda i: (i, 0)),
            pl.BlockSpec((BN, D), lambda i: (i, 0)),
            pl.BlockSpec((D, no), lambda i: (0, 0)),
            pl.BlockSpec((1, no), lambda i: (0, 0)),
        ],
        out_specs=pl.BlockSpec((BN, no), lambda i: (i, 0)),
        out_shape=jax.ShapeDtypeStruct((N, no), jnp.float32),
    )(p0, p1, hprev, wh, bh)


def kernel(x, edge_index, rel_type, norm, W1, W2, W3, Wa, ba, Wb, bb):
    src = edge_index[0].astype(jnp.int32)
    dst = edge_index[1].astype(jnp.int32)
    rel = rel_type.astype(jnp.int32)
    gidx = src * R + rel
    pad = EPAD + ESLACK - E
    zi = jnp.zeros((pad,), jnp.int32)
    gidx2 = jnp.concatenate([gidx, zi]).reshape((EPAD + ESLACK) // 64, 64)
    dst2 = jnp.concatenate([dst, zi]).reshape((EPAD + ESLACK) // 64, 64)
    normp = jnp.concatenate([norm, jnp.zeros((pad,), jnp.float32)])

    wc1 = W1.transpose(1, 0, 2).reshape(D, R * D)
    wc2 = W2.transpose(1, 0, 2).reshape(D, R * D)
    wc3 = W3.transpose(1, 0, 2).reshape(D, R * D)
    wh = jnp.concatenate([Wa.T, Wb.T], axis=1)
    bh = jnp.concatenate([ba, bb]).reshape(1, OUT_A + OUT_B)

    xw1 = _tc_transform(x, wc1)
    p1 = _sc_edge(xw1.reshape(R * N, D), gidx2, dst2, normp)
    h1, xw2 = _tc_combine(p1[:N], p1[NPAD:NPAD + N], None, wc2)
    p2 = _sc_edge(xw2.reshape(R * N, D), gidx2, dst2, normp)
    h2, xw3 = _tc_combine(p2[:N], p2[NPAD:NPAD + N], h1, wc3)
    p3 = _sc_edge(xw3.reshape(R * N, D), gidx2, dst2, normp)
    out = _tc_head(p3[:N], p3[NPAD:NPAD + N], h2, wh, bh)
    return out[:, :OUT_A], out[:, OUT_A:]


# speedup vs baseline: 1.1366x; 1.0860x over previous
import functools

import jax
import jax.numpy as jnp
from jax import lax
from jax.experimental import pallas as pl
from jax.experimental.pallas import tpu as pltpu
from jax.experimental.pallas import tpu_sc as plsc

N = 100000
E = 3200000
D = 16
R = 3
OUT_A = 2
OUT_B = 16

NC = 2
NS = 16
NT = NC * NS
CHUNK = 768
RPC = CHUNK // 64
NCH_A = 90
NCH_B = 172
NCHT = 16 * (NCH_A + NCH_B)
EPAD = NCHT * CHUNK
ESLACK = 0
NPAD = 100096
NPT = NPAD // NS

BN = 10000


def _sc_edge_body(table, gidx, dstx, normx, out, acc,
                  i0, i1, d0, d1, n0, n1,
                  rows0, rows1, lsem, g0, g1, g2, g3, ssem):
    c = lax.axis_index("c")
    s = lax.axis_index("s")
    ncw = NCH_A + c * (NCH_B - NCH_A)
    cbase = (1 - c) * (s * NCH_A) + c * (NS * NCH_A + s * NCH_B)
    idxr = (i0, i1)
    dstr = (d0, d1)
    normr = (n0, n1)
    rowsr = (rows0, rows1)
    gsems = (g0, g1, g2, g3)

    def fire_linear(k, pr):
        r0 = (cbase + k) * RPC
        e0 = (cbase + k) * CHUNK
        pltpu.async_copy(gidx.at[pl.ds(r0, RPC)], idxr[pr], lsem)
        pltpu.async_copy(dstx.at[pl.ds(r0, RPC)], dstr[pr], lsem)
        pltpu.async_copy(normx.at[pl.ds(e0, CHUNK)], normr[pr], lsem)

    def wait_linear(k, pr):
        r0 = (cbase + k) * RPC
        e0 = (cbase + k) * CHUNK
        pltpu.make_async_copy(gidx.at[pl.ds(r0, RPC)], idxr[pr], lsem).wait()
        pltpu.make_async_copy(dstx.at[pl.ds(r0, RPC)], dstr[pr], lsem).wait()
        pltpu.make_async_copy(normx.at[pl.ds(e0, CHUNK)], normr[pr],
                              lsem).wait()

    QS = RPC // 4

    def fire_gather(pr):
        for j in range(RPC):
            pltpu.async_copy(table.at[idxr[pr].at[j]],
                             rowsr[pr].at[pl.ds(j * 64, 64)], gsems[j // QS])

    def wait_gather_q(pr, q):
        for j in range(q * QS, (q + 1) * QS):
            pltpu.make_async_copy(table.at[idxr[pr].at[j]],
                                  rowsr[pr].at[pl.ds(j * 64, 64)],
                                  gsems[j // QS]).wait()

    def scale_q(pr, q):
        sub = QS * 64 // 16

        @pl.loop(0, sub)
        def _scale(g):
            e0 = q * QS * 64 + g * 16
            nv = normr[pr][pl.ds(e0, 16)]
            for j in range(16):
                e = e0 + j
                rowsr[pr][e, :] = rowsr[pr][e, :] * nv[j]

    def wait_scale_gather(pr):
        for q in range(4):
            wait_gather_q(pr, q)
            scale_q(pr, q)

    def fire_scatter(pr):
        for j in range(RPC):
            pltpu.async_copy(rowsr[pr].at[pl.ds(j * 64, 64)],
                             acc.at[dstr[pr].at[j]], ssem, add=True)

    def wait_scatter(pr):
        for j in range(RPC):
            pltpu.make_async_copy(rowsr[pr].at[pl.ds(j * 64, 64)],
                                  acc.at[dstr[pr].at[j]], ssem).wait()

    @pl.loop(0, CHUNK)
    def _zero(i):
        rows0[i, :] = jnp.zeros((D,), jnp.float32)

    @pl.loop(0, 8)
    def _zacc(k):
        pltpu.sync_copy(rows0, acc.at[pl.ds(s * NPT + k * CHUNK, CHUNK)])
    pltpu.sync_copy(rows0.at[pl.ds(0, NPT - 8 * CHUNK)],
                    acc.at[pl.ds(s * NPT + 8 * CHUNK, NPT - 8 * CHUNK)])

    plsc.subcore_barrier()

    def sub(i, pr):
        wait_linear(i, pr)
        fire_gather(pr)
        wait_scatter(1 - pr)
        fire_linear(i + 1, 1 - pr)
        wait_scale_gather(pr)
        fire_scatter(pr)

    fire_linear(0, 0)
    wait_linear(0, 0)
    fire_gather(0)
    fire_linear(1, 1)
    wait_scale_gather(0)
    fire_scatter(0)

    @pl.loop(0, (ncw - 2) // 2)
    def _mid(t):
        sub(1 + 2 * t, 1)
        sub(2 + 2 * t, 0)

    wait_linear(ncw - 1, 1)
    fire_gather(1)
    wait_scatter(0)
    wait_scale_gather(1)
    fire_scatter(1)
    wait_scatter(1)

    plsc.subcore_barrier()
    pltpu.sync_copy(acc.at[pl.ds(s * NPT, NPT)],
                    out.at[pl.ds(c * NPAD + s * NPT, NPT)])


_sc_edge = pl.kernel(
    _sc_edge_body,
    out_type=jax.ShapeDtypeStruct((2 * NPAD, D), jnp.float32),
    mesh=plsc.VectorSubcoreMesh(core_axis_name="c", subcore_axis_name="s",
                                num_cores=NC, num_subcores=NS),
    scratch_types=[
        pltpu.MemorySpace.VMEM_SHARED((NPAD, D), jnp.float32),
        *[pltpu.VMEM((RPC, 64), jnp.int32) for _ in range(2)],
        *[pltpu.VMEM((RPC, 64), jnp.int32) for _ in range(2)],
        *[pltpu.VMEM((CHUNK,), jnp.float32) for _ in range(2)],
        *[pltpu.VMEM((CHUNK, D), jnp.float32) for _ in range(2)],
        *[pltpu.SemaphoreType.DMA for _ in range(6)],
    ],
    compiler_params=pltpu.CompilerParams(use_tc_tiling_on_sc=False),
)


def _transform_body(x_ref, w_ref, xw_ref):
    xw_ref[...] = jnp.dot(x_ref[...], w_ref[...],
                          preferred_element_type=jnp.float32)


def _tc_transform(x, wc):
    return pl.pallas_call(
        _transform_body,
        grid=(N // BN,),
        in_specs=[
            pl.BlockSpec((BN, D), lambda i: (i, 0)),
            pl.BlockSpec((D, R * D), lambda i: (0, 0)),
        ],
        out_specs=pl.BlockSpec((BN, R * D), lambda i: (i, 0)),
        out_shape=jax.ShapeDtypeStruct((N, R * D), jnp.float32),
    )(x, wc)


def _combine_body(with_skip, *refs):
    if with_skip:
        p0_ref, p1_ref, h_ref, w_ref, hn_ref, xw_ref = refs
        h = jnp.maximum(p0_ref[...] + p1_ref[...] + h_ref[...], 0.0)
    else:
        p0_ref, p1_ref, w_ref, hn_ref, xw_ref = refs
        h = jnp.maximum(p0_ref[...] + p1_ref[...], 0.0)
    hn_ref[...] = h
    xw_ref[...] = jnp.dot(h, w_ref[...], preferred_element_type=jnp.float32)


def _tc_combine(p0, p1, hprev, wc):
    with_skip = hprev is not None
    hb = [pl.BlockSpec((BN, D), lambda i: (i, 0))] if with_skip else []
    ops = (p0, p1) + ((hprev,) if with_skip else ()) + (wc,)
    return pl.pallas_call(
        functools.partial(_combine_body, with_skip),
        grid=(N // BN,),
        in_specs=[
            pl.BlockSpec((BN, D), lambda i: (i, 0)),
            pl.BlockSpec((BN, D), lambda i: (i, 0)),
            *hb,
            pl.BlockSpec((D, R * D), lambda i: (0, 0)),
        ],
        out_specs=[
            pl.BlockSpec((BN, D), lambda i: (i, 0)),
            pl.BlockSpec((BN, R * D), lambda i: (i, 0)),
        ],
        out_shape=[
            jax.ShapeDtypeStruct((N, D), jnp.float32),
            jax.ShapeDtypeStruct((N, R * D), jnp.float32),
        ],
    )(*ops)


def _head_body(p0_ref, p1_ref, h_ref, w_ref, b_ref, o_ref):
    h = jnp.maximum(p0_ref[...] + p1_ref[...] + h_ref[...], 0.0)
    o_ref[...] = jnp.dot(h, w_ref[...],
                         preferred_element_type=jnp.float32) + b_ref[...]


def _tc_head(p0, p1, hprev, wh, bh):
    no = OUT_A + OUT_B
    return pl.pallas_call(
        _head_body,
        grid=(N // BN,),
        in_specs=[
            pl.BlockSpec((BN, D), lambda i: (i, 0)),
            pl.BlockSpec((BN, D), lambda i: (i, 0)),
            pl.BlockSpec((BN, D), lambda i: (i, 0)),
            pl.BlockSpec((D, no), lambda i: (0, 0)),
            pl.BlockSpec((1, no), lambda i: (0, 0)),
        ],
        out_specs=pl.BlockSpec((BN, no), lambda i: (i, 0)),
        out_shape=jax.ShapeDtypeStruct((N, no), jnp.float32),
    )(p0, p1, hprev, wh, bh)


def kernel(x, edge_index, rel_type, norm, W1, W2, W3, Wa, ba, Wb, bb):
    src = edge_index[0].astype(jnp.int32)
    dst = edge_index[1].astype(jnp.int32)
    rel = rel_type.astype(jnp.int32)
    gidx = src * R + rel
    pad = EPAD + ESLACK - E
    zi = jnp.zeros((pad,), jnp.int32)
    gidx2 = jnp.concatenate([gidx, zi]).reshape((EPAD + ESLACK) // 64, 64)
    dst2 = jnp.concatenate([dst, zi]).reshape((EPAD + ESLACK) // 64, 64)
    normp = jnp.concatenate([norm, jnp.zeros((pad,), jnp.float32)])

    wc1 = W1.transpose(1, 0, 2).reshape(D, R * D)
    wc2 = W2.transpose(1, 0, 2).reshape(D, R * D)
    wc3 = W3.transpose(1, 0, 2).reshape(D, R * D)
    wh = jnp.concatenate([Wa.T, Wb.T], axis=1)
    bh = jnp.concatenate([ba, bb]).reshape(1, OUT_A + OUT_B)

    xw1 = _tc_transform(x, wc1)
    p1 = _sc_edge(xw1.reshape(R * N, D), gidx2, dst2, normp)
    h1, xw2 = _tc_combine(p1[:N], p1[NPAD:NPAD + N], None, wc2)
    p2 = _sc_edge(xw2.reshape(R * N, D), gidx2, dst2, normp)
    h2, xw3 = _tc_combine(p2[:N], p2[NPAD:NPAD + N], h1, wc3)
    p3 = _sc_edge(xw3.reshape(R * N, D), gidx2, dst2, normp)
    out = _tc_head(p3[:N], p3[NPAD:NPAD + N], h2, wh, bh)
    return out[:, :OUT_A], out[:, OUT_A:]
